# baseline (speedup 1.0000x reference)
import functools

import jax
import jax.numpy as jnp
import numpy as np
from jax import lax
from jax.experimental import pallas as pl
from jax.experimental.pallas import tpu as pltpu
from jax.experimental.pallas import tpu_sc as plsc

_F_SAMPLE = 250.0
_FREQ = 0.5
_AMPLITUDE = 0.05
_BLOCK_ROWS = 1024
_BLOCK_COLS = 2048
_N_SEL = 64
_NC = 2
_NS = 16


def _osc_vec_kernel(phase_ref, osc_ref):
    t_len = osc_ref.shape[1]
    col = jax.lax.broadcasted_iota(jnp.int32, (1, t_len), 1).astype(jnp.float32)
    step = (t_len / _F_SAMPLE) / (t_len - 1)
    osc_ref[...] = _AMPLITUDE * jnp.sin(
        (2.0 * np.pi * _FREQ * step) * col + phase_ref[0]
    )


def _copy_kernel(data_ref, out_ref):
    out_ref[...] = data_ref[...]


def _merge_kernel(sel_ref, upd_ref, cpy_ref, out_ref):
    del sel_ref, cpy_ref
    out_ref[...] = upd_ref[...]


def _sc_update(data_hbm, sel_hbm, osc_hbm, out_hbm, idx_v, osc_v, rows_v):
    w = lax.axis_index("s") * _NC + lax.axis_index("c")
    pltpu.sync_copy(sel_hbm, idx_v)
    pltpu.sync_copy(osc_hbm, osc_v)
    iota = lax.broadcasted_iota(jnp.int32, (16,), 0)
    p0 = 2 * w
    p1 = p0 + 1
    r0 = jnp.int32(-1)
    r1 = jnp.int32(-1)
    for c in range(_N_SEL // 16):
        vec = idx_v[pl.ds(16 * c, 16)]
        pos = 16 * c + iota
        r0 = jnp.maximum(r0, jnp.max(jnp.where(pos == p0, vec, -1)))
        r1 = jnp.maximum(r1, jnp.max(jnp.where(pos == p1, vec, -1)))
    pltpu.sync_copy(data_hbm.at[r0], rows_v.at[0])
    pltpu.sync_copy(data_hbm.at[r1], rows_v.at[1])

    def body(i, carry):
        s = pl.ds(i * 16, 16)
        rows_v[0, s] = rows_v[0, s] + osc_v[s]
        rows_v[1, s] = rows_v[1, s] + osc_v[s]
        return carry

    lax.fori_loop(0, rows_v.shape[1] // 16, body, 0)
    pltpu.sync_copy(rows_v.at[0], out_hbm.at[p0])
    pltpu.sync_copy(rows_v.at[1], out_hbm.at[p1])


def kernel(data, selection, phase):
    n_ts, t_len = data.shape
    sel = selection.astype(jnp.int32)
    phase_arr = jnp.reshape(phase, (1,)).astype(jnp.float32)

    osc = pl.pallas_call(
        _osc_vec_kernel,
        in_specs=[pl.BlockSpec(memory_space=pltpu.SMEM)],
        out_specs=pl.BlockSpec((1, t_len), lambda: (0, 0)),
        out_shape=jax.ShapeDtypeStruct((1, t_len), jnp.float32),
    )(phase_arr).reshape(t_len)

    sc_update = functools.partial(
        pl.kernel,
        out_type=jax.ShapeDtypeStruct((_N_SEL, t_len), jnp.float32),
        mesh=plsc.VectorSubcoreMesh(core_axis_name="c", subcore_axis_name="s"),
        scratch_types=[
            pltpu.VMEM((_N_SEL,), jnp.int32),
            pltpu.VMEM((t_len,), jnp.float32),
            pltpu.VMEM((2, t_len), jnp.float32),
        ],
        compiler_params=pltpu.CompilerParams(needs_layout_passes=False),
    )(_sc_update)
    upd = sc_update(data, sel, osc)

    cpy = pl.pallas_call(
        _copy_kernel,
        grid=(n_ts // _BLOCK_ROWS, t_len // _BLOCK_COLS),
        in_specs=[
            pl.BlockSpec((_BLOCK_ROWS, _BLOCK_COLS), lambda i, j: (i, j)),
        ],
        out_specs=pl.BlockSpec((_BLOCK_ROWS, _BLOCK_COLS), lambda i, j: (i, j)),
        out_shape=jax.ShapeDtypeStruct((n_ts, t_len), jnp.float32),
        compiler_params=pltpu.CompilerParams(
            dimension_semantics=("arbitrary", "arbitrary"),
        ),
    )(data)

    grid_spec = pltpu.PrefetchScalarGridSpec(
        num_scalar_prefetch=1,
        grid=(_N_SEL,),
        in_specs=[
            pl.BlockSpec((1, 1, t_len), lambda i, sel_ref: (i, 0, 0)),
            pl.BlockSpec(memory_space=pl.ANY),
        ],
        out_specs=pl.BlockSpec(
            (1, 1, t_len), lambda i, sel_ref: (sel_ref[i], 0, 0)
        ),
    )
    out = pl.pallas_call(
        _merge_kernel,
        grid_spec=grid_spec,
        out_shape=jax.ShapeDtypeStruct((n_ts, 1, t_len), jnp.float32),
        input_output_aliases={2: 0},
        compiler_params=pltpu.CompilerParams(
            dimension_semantics=("arbitrary",),
        ),
    )(sel, upd.reshape(_N_SEL, 1, t_len), cpy.reshape(n_ts, 1, t_len))
    return out.reshape(n_ts, t_len)


# speedup vs baseline: 4.2780x; 4.2780x over previous
import functools

import jax
import jax.numpy as jnp
import numpy as np
from jax import lax
from jax.experimental import pallas as pl
from jax.experimental.pallas import tpu as pltpu
from jax.experimental.pallas import tpu_sc as plsc

_F_SAMPLE = 250.0
_FREQ = 0.5
_AMPLITUDE = 0.05
_BLOCK_ROWS = 1024
_BLOCK_COLS = 2048
_N_SEL = 64
_NC = 2
_NS = 16


def _osc_vec_kernel(phase_ref, osc_ref):
    t_len = osc_ref.shape[1]
    col = jax.lax.broadcasted_iota(jnp.int32, (1, t_len), 1).astype(jnp.float32)
    step = (t_len / _F_SAMPLE) / (t_len - 1)
    osc_ref[...] = _AMPLITUDE * jnp.sin(
        (2.0 * np.pi * _FREQ * step) * col + phase_ref[0]
    )


def _copy_kernel(data_ref, out_ref):
    out_ref[...] = data_ref[...]


def _merge_kernel(sel_ref, upd_ref, cpy_ref, out_ref, sem):
    del cpy_ref
    copies = []
    for i in range(_N_SEL):
        cp = pltpu.make_async_copy(
            upd_ref.at[pl.ds(i, 1), :],
            out_ref.at[pl.ds(sel_ref[i], 1), :],
            sem,
        )
        cp.start()
        copies.append(cp)
    for cp in copies:
        cp.wait()


def _sc_update(data_hbm, sel_hbm, osc_hbm, out_hbm, idx_v, osc_v, rows_v):
    w = lax.axis_index("s") * _NC + lax.axis_index("c")
    pltpu.sync_copy(sel_hbm, idx_v)
    pltpu.sync_copy(osc_hbm, osc_v)
    iota = lax.broadcasted_iota(jnp.int32, (16,), 0)
    p0 = 2 * w
    p1 = p0 + 1
    r0 = jnp.int32(-1)
    r1 = jnp.int32(-1)
    for c in range(_N_SEL // 16):
        vec = idx_v[pl.ds(16 * c, 16)]
        pos = 16 * c + iota
        r0 = jnp.maximum(r0, jnp.max(jnp.where(pos == p0, vec, -1)))
        r1 = jnp.maximum(r1, jnp.max(jnp.where(pos == p1, vec, -1)))
    pltpu.sync_copy(data_hbm.at[r0], rows_v.at[0])
    pltpu.sync_copy(data_hbm.at[r1], rows_v.at[1])

    def body(i, carry):
        s = pl.ds(i * 16, 16)
        rows_v[0, s] = rows_v[0, s] + osc_v[s]
        rows_v[1, s] = rows_v[1, s] + osc_v[s]
        return carry

    lax.fori_loop(0, rows_v.shape[1] // 16, body, 0)
    pltpu.sync_copy(rows_v.at[0], out_hbm.at[p0])
    pltpu.sync_copy(rows_v.at[1], out_hbm.at[p1])


def kernel(data, selection, phase):
    n_ts, t_len = data.shape
    sel = selection.astype(jnp.int32)
    phase_arr = jnp.reshape(phase, (1,)).astype(jnp.float32)

    osc = pl.pallas_call(
        _osc_vec_kernel,
        in_specs=[pl.BlockSpec(memory_space=pltpu.SMEM)],
        out_specs=pl.BlockSpec((1, t_len), lambda: (0, 0)),
        out_shape=jax.ShapeDtypeStruct((1, t_len), jnp.float32),
    )(phase_arr).reshape(t_len)

    sc_update = functools.partial(
        pl.kernel,
        out_type=jax.ShapeDtypeStruct((_N_SEL, t_len), jnp.float32),
        mesh=plsc.VectorSubcoreMesh(core_axis_name="c", subcore_axis_name="s"),
        scratch_types=[
            pltpu.VMEM((_N_SEL,), jnp.int32),
            pltpu.VMEM((t_len,), jnp.float32),
            pltpu.VMEM((2, t_len), jnp.float32),
        ],
        compiler_params=pltpu.CompilerParams(needs_layout_passes=False),
    )(_sc_update)
    upd = sc_update(data, sel, osc)

    cpy = pl.pallas_call(
        _copy_kernel,
        grid=(n_ts // _BLOCK_ROWS, t_len // _BLOCK_COLS),
        in_specs=[
            pl.BlockSpec((_BLOCK_ROWS, _BLOCK_COLS), lambda i, j: (i, j)),
        ],
        out_specs=pl.BlockSpec((_BLOCK_ROWS, _BLOCK_COLS), lambda i, j: (i, j)),
        out_shape=jax.ShapeDtypeStruct((n_ts, t_len), jnp.float32),
        compiler_params=pltpu.CompilerParams(
            dimension_semantics=("arbitrary", "arbitrary"),
        ),
    )(data)

    out = pl.pallas_call(
        _merge_kernel,
        in_specs=[
            pl.BlockSpec(memory_space=pltpu.SMEM),
            pl.BlockSpec(memory_space=pltpu.VMEM),
            pl.BlockSpec(memory_space=pl.ANY),
        ],
        out_specs=pl.BlockSpec(memory_space=pl.ANY),
        out_shape=jax.ShapeDtypeStruct((n_ts, t_len), jnp.float32),
        input_output_aliases={2: 0},
        scratch_shapes=[pltpu.SemaphoreType.DMA],
    )(sel, upd, cpy)
    return out


# speedup vs baseline: 4.7183x; 1.1029x over previous
import jax
import jax.numpy as jnp
import numpy as np
from jax.experimental import pallas as pl
from jax.experimental.pallas import tpu as pltpu

_F_SAMPLE = 250.0
_FREQ = 0.5
_AMPLITUDE = 0.05
_BLOCK_ROWS = 512
_BLOCK_COLS = 2048


def _osc_kernel(sel_ref, phase_ref, data_ref, out_ref):
    i = pl.program_id(0)
    j = pl.program_id(1)
    br, bc = data_ref.shape
    rows = i * br + jax.lax.broadcasted_iota(jnp.int32, (br, 1), 0)
    sel = sel_ref[0, :]
    hit = (rows == sel[None, :]).any(axis=1, keepdims=True)
    col = (j * bc + jax.lax.broadcasted_iota(jnp.int32, (1, bc), 1)).astype(
        jnp.float32
    )
    step = (4096.0 / _F_SAMPLE) / (4096.0 - 1.0)
    osc = _AMPLITUDE * jnp.sin(
        (2.0 * np.pi * _FREQ * step) * col + phase_ref[0]
    )
    out_ref[...] = data_ref[...] + jnp.where(hit, osc, 0.0)


def kernel(data, selection, phase):
    n_ts, t_len = data.shape
    sel2 = selection.astype(jnp.int32).reshape(1, -1)
    phase_arr = jnp.reshape(phase, (1,)).astype(jnp.float32)
    grid = (n_ts // _BLOCK_ROWS, t_len // _BLOCK_COLS)
    return pl.pallas_call(
        _osc_kernel,
        grid=grid,
        in_specs=[
            pl.BlockSpec((1, sel2.shape[1]), lambda i, j: (0, 0)),
            pl.BlockSpec(memory_space=pltpu.SMEM),
            pl.BlockSpec((_BLOCK_ROWS, _BLOCK_COLS), lambda i, j: (i, j)),
        ],
        out_specs=pl.BlockSpec((_BLOCK_ROWS, _BLOCK_COLS), lambda i, j: (i, j)),
        out_shape=jax.ShapeDtypeStruct((n_ts, t_len), jnp.float32),
        compiler_params=pltpu.CompilerParams(
            dimension_semantics=("arbitrary", "arbitrary"),
        ),
    )(sel2, phase_arr, data)


# speedup vs baseline: 4.7983x; 1.0170x over previous
import jax
import jax.numpy as jnp
import numpy as np
from jax.experimental import pallas as pl
from jax.experimental.pallas import tpu as pltpu

_F_SAMPLE = 250.0
_FREQ = 0.5
_AMPLITUDE = 0.05
_BLOCK_ROWS = 1024
_BLOCK_COLS = 2048


def _osc_kernel(sel_ref, phase_ref, data_ref, out_ref):
    i = pl.program_id(0)
    j = pl.program_id(1)
    br, bc = data_ref.shape
    rows = i * br + jax.lax.broadcasted_iota(jnp.int32, (br, 1), 0)
    sel = sel_ref[0, :]
    hit = (rows == sel[None, :]).any(axis=1, keepdims=True)
    col = (j * bc + jax.lax.broadcasted_iota(jnp.int32, (1, bc), 1)).astype(
        jnp.float32
    )
    step = (4096.0 / _F_SAMPLE) / (4096.0 - 1.0)
    osc = _AMPLITUDE * jnp.sin(
        (2.0 * np.pi * _FREQ * step) * col + phase_ref[0]
    )
    out_ref[...] = data_ref[...] + jnp.where(hit, osc, 0.0)


def kernel(data, selection, phase):
    n_ts, t_len = data.shape
    sel2 = selection.astype(jnp.int32).reshape(1, -1)
    phase_arr = jnp.reshape(phase, (1,)).astype(jnp.float32)
    grid = (n_ts // _BLOCK_ROWS, t_len // _BLOCK_COLS)
    return pl.pallas_call(
        _osc_kernel,
        grid=grid,
        in_specs=[
            pl.BlockSpec((1, sel2.shape[1]), lambda i, j: (0, 0)),
            pl.BlockSpec(memory_space=pltpu.SMEM),
            pl.BlockSpec((_BLOCK_ROWS, _BLOCK_COLS), lambda i, j: (i, j)),
        ],
        out_specs=pl.BlockSpec((_BLOCK_ROWS, _BLOCK_COLS), lambda i, j: (i, j)),
        out_shape=jax.ShapeDtypeStruct((n_ts, t_len), jnp.float32),
        compiler_params=pltpu.CompilerParams(
            dimension_semantics=("arbitrary", "arbitrary"),
        ),
    )(sel2, phase_arr, data)
